# baseline (device time: 443799 ns/iter reference)
import jax
import jax.numpy as jnp
from jax import lax
from jax.experimental import pallas as pl
from jax.experimental.pallas import tpu as pltpu

N = 8192
NBH = 16
BN = (N // 2) // NBH
S = 2048
S_HALF = S // 2
NSLOT = 3


def kernel(O, Wo):
    _, s, h, d = O.shape
    k, n = Wo.shape
    assert (s, h * d, n) == (S, k, N)
    O2 = O.reshape(S, k)

    def body(o_ref, wo_ref, out_ref, acc_ref, wbuf_ref, ysend_ref, yrecv_ref,
             sum_ref, xrecv_ref, ysend_sems, yrecv_sems, xsend_sems,
             xrecv_sems, wdma_sems, outd_sems, xout_sems):
        j = pl.program_id(0)
        my_x = lax.axis_index("x")
        my_y = lax.axis_index("y")
        ypeer = (my_x, 1 - my_y)
        xpeer = (1 - my_x, my_y)
        my_start = my_y * S_HALF
        other_start = (1 - my_y) * S_HALF
        my_col0 = my_x * (N // 2)
        other_col0 = (1 - my_x) * (N // 2)

        def y_rdma(c):
            return pltpu.make_async_remote_copy(
                src_ref=ysend_ref.at[c], dst_ref=yrecv_ref.at[c],
                send_sem=ysend_sems.at[c], recv_sem=yrecv_sems.at[c],
                device_id=ypeer, device_id_type=pl.DeviceIdType.MESH)

        def x_rdma(c):
            return pltpu.make_async_remote_copy(
                src_ref=sum_ref.at[c], dst_ref=xrecv_ref.at[c],
                send_sem=xsend_sems.at[c], recv_sem=xrecv_sems.at[c],
                device_id=xpeer, device_id_type=pl.DeviceIdType.MESH)

        def w_copy(slot, blk):
            return pltpu.make_async_copy(
                wo_ref.at[:, pl.ds(my_col0 + blk * BN, BN)],
                wbuf_ref.at[slot], wdma_sems.at[slot])

        def outd_copy(c, b):
            return pltpu.make_async_copy(
                sum_ref.at[c],
                out_ref.at[0, :, pl.ds(my_col0 + b * BN, BN)],
                outd_sems.at[c])

        def xout_copy(c, b):
            return pltpu.make_async_copy(
                xrecv_ref.at[c],
                out_ref.at[0, :, pl.ds(other_col0 + b * BN, BN)],
                xout_sems.at[c])

        @pl.when(j == 0)
        def _():
            barrier = pltpu.get_barrier_semaphore()
            for nbr in (ypeer, xpeer):
                pl.semaphore_signal(barrier, inc=1, device_id=nbr,
                                    device_id_type=pl.DeviceIdType.MESH)
            pl.semaphore_wait(barrier, 2)
            w_copy(0, 0).start()

        @pl.when(j < NBH)
        def _():
            for wslot in range(2):
                @pl.when(lax.rem(j, 2) == wslot)
                def _(wslot=wslot):
                    w_copy(wslot, j).wait()
                    @pl.when(j + 1 < NBH)
                    def _():
                        w_copy(1 - wslot, j + 1).start()
                    for c in range(NSLOT):
                        @pl.when(lax.rem(j, NSLOT) == c)
                        def _(c=c):
                            @pl.when(j >= NSLOT)
                            def _():
                                y_rdma(c).wait_send()
                            ysend_ref[c] = jnp.dot(
                                o_ref[pl.ds(other_start, S_HALF), :],
                                wbuf_ref[wslot],
                                preferred_element_type=jnp.float32)
                            y_rdma(c).start()
                    acc_ref[wslot] = jnp.dot(
                        o_ref[pl.ds(my_start, S_HALF), :], wbuf_ref[wslot],
                        preferred_element_type=jnp.float32)

        @pl.when(j >= 2)
        def _():
            cblk = j - 2
            @pl.when(j >= 3)
            def _():
                for c in range(NSLOT):
                    @pl.when(lax.rem(j, NSLOT) == c)
                    def _(c=c):
                        xout_copy(c, 0).wait()
            for c in range(NSLOT):
                @pl.when(lax.rem(cblk, NSLOT) == c)
                def _(c=c):
                    x_rdma(c).wait_recv()
                    xout_copy(c, cblk).start()

        @pl.when((j >= 1) & (j <= NBH))
        def _():
            b = j - 1
            for c in range(NSLOT):
                @pl.when(lax.rem(b, NSLOT) == c)
                def _(c=c):
                    @pl.when(b >= NSLOT)
                    def _():
                        x_rdma(c).wait_send()
                        outd_copy(c, 0).wait()
                    y_rdma(c).wait_recv()
                    for aslot in range(2):
                        @pl.when(lax.rem(b, 2) == aslot)
                        def _(aslot=aslot):
                            sum_ref[c] = acc_ref[aslot] + yrecv_ref[c]
                    outd_copy(c, b).start()
                    x_rdma(c).start()

        @pl.when(j == NBH + 1)
        def _():
            for c in range(NSLOT):
                y_rdma(c).wait_send()
                x_rdma(c).wait_send()
                outd_copy(c, 0).wait()
            xout_copy(lax.rem(NBH - 1, NSLOT), 0).wait()

    return pl.pallas_call(
        body,
        grid=(NBH + 2,),
        in_specs=[
            pl.BlockSpec((S, k), lambda j: (0, 0)),
            pl.BlockSpec(memory_space=pl.ANY),
        ],
        out_specs=pl.BlockSpec(memory_space=pl.ANY),
        out_shape=jax.ShapeDtypeStruct((1, S_HALF, n), jnp.float32),
        scratch_shapes=[
            pltpu.VMEM((2, S_HALF, BN), jnp.float32),
            pltpu.VMEM((2, k, BN), jnp.float32),
            pltpu.VMEM((NSLOT, S_HALF, BN), jnp.float32),
            pltpu.VMEM((NSLOT, S_HALF, BN), jnp.float32),
            pltpu.VMEM((NSLOT, S_HALF, BN), jnp.float32),
            pltpu.VMEM((NSLOT, S_HALF, BN), jnp.float32),
            pltpu.SemaphoreType.DMA((NSLOT,)),
            pltpu.SemaphoreType.DMA((NSLOT,)),
            pltpu.SemaphoreType.DMA((NSLOT,)),
            pltpu.SemaphoreType.DMA((NSLOT,)),
            pltpu.SemaphoreType.DMA((2,)),
            pltpu.SemaphoreType.DMA((NSLOT,)),
            pltpu.SemaphoreType.DMA((NSLOT,)),
        ],
        compiler_params=pltpu.CompilerParams(
            collective_id=0,
            dimension_semantics=("arbitrary",),
            vmem_limit_bytes=64 * 1024 * 1024,
        ),
    )(O2, Wo)


# device time: 262303 ns/iter; 1.6919x vs baseline; 1.6919x over previous
import jax
import jax.numpy as jnp
from jax import lax
from jax.experimental import pallas as pl
from jax.experimental.pallas import tpu as pltpu

N = 8192
NBH = 16
BN = (N // 2) // NBH
S = 2048
S_HALF = S // 2
NSLOT = 3


def kernel(O, Wo):
    _, s, nh, d = O.shape
    k, n = Wo.shape
    assert (s, nh * d, n) == (S, k, N)

    def body(o_ref, wo_ref, out_ref, o2_ref, acc_ref, wbuf_ref, ysend_ref,
             yrecv_ref, sum_ref, xrecv_ref, odma_sems, ysend_sems,
             yrecv_sems, xsend_sems, xrecv_sems, wdma_sems, outd_sems,
             xout_sems):
        j = pl.program_id(0)
        my_x = lax.axis_index("x")
        my_y = lax.axis_index("y")
        ypeer = (my_x, 1 - my_y)
        xpeer = (1 - my_x, my_y)
        my_start = my_y * S_HALF
        other_start = (1 - my_y) * S_HALF
        my_col0 = my_x * (N // 2)
        other_col0 = (1 - my_x) * (N // 2)

        def y_rdma(c):
            return pltpu.make_async_remote_copy(
                src_ref=ysend_ref.at[c], dst_ref=yrecv_ref.at[c],
                send_sem=ysend_sems.at[c], recv_sem=yrecv_sems.at[c],
                device_id=ypeer, device_id_type=pl.DeviceIdType.MESH)

        def x_rdma(c):
            return pltpu.make_async_remote_copy(
                src_ref=sum_ref.at[c], dst_ref=xrecv_ref.at[c],
                send_sem=xsend_sems.at[c], recv_sem=xrecv_sems.at[c],
                device_id=xpeer, device_id_type=pl.DeviceIdType.MESH)

        def w_copy(slot, blk):
            return pltpu.make_async_copy(
                wo_ref.at[:, pl.ds(my_col0 + blk * BN, BN)],
                wbuf_ref.at[slot], wdma_sems.at[slot])

        def outd_copy(c, b):
            return pltpu.make_async_copy(
                sum_ref.at[c],
                out_ref.at[0, :, pl.ds(my_col0 + b * BN, BN)],
                outd_sems.at[c])

        def xout_copy(c, b):
            return pltpu.make_async_copy(
                xrecv_ref.at[c],
                out_ref.at[0, :, pl.ds(other_col0 + b * BN, BN)],
                xout_sems.at[c])

        def o_copy(hh):
            return pltpu.make_async_copy(
                o_ref.at[0, :, hh, :],
                o2_ref.at[:, pl.ds(hh * d, d)],
                odma_sems.at[hh])

        @pl.when(j == 0)
        def _():
            for hh in range(nh):
                o_copy(hh).start()
            barrier = pltpu.get_barrier_semaphore()
            for nbr in (ypeer, xpeer):
                pl.semaphore_signal(barrier, inc=1, device_id=nbr,
                                    device_id_type=pl.DeviceIdType.MESH)
            pl.semaphore_wait(barrier, 2)
            w_copy(0, 0).start()
            for hh in range(nh):
                o_copy(hh).wait()

        @pl.when(j < NBH)
        def _():
            for wslot in range(2):
                @pl.when(lax.rem(j, 2) == wslot)
                def _(wslot=wslot):
                    w_copy(wslot, j).wait()
                    @pl.when(j + 1 < NBH)
                    def _():
                        w_copy(1 - wslot, j + 1).start()
                    for c in range(NSLOT):
                        @pl.when(lax.rem(j, NSLOT) == c)
                        def _(c=c):
                            @pl.when(j >= NSLOT)
                            def _():
                                y_rdma(c).wait_send()
                            ysend_ref[c] = jnp.dot(
                                o2_ref[pl.ds(other_start, S_HALF), :],
                                wbuf_ref[wslot],
                                preferred_element_type=jnp.float32)
                            y_rdma(c).start()
                    acc_ref[wslot] = jnp.dot(
                        o2_ref[pl.ds(my_start, S_HALF), :], wbuf_ref[wslot],
                        preferred_element_type=jnp.float32)

        @pl.when(j >= 2)
        def _():
            cblk = j - 2
            @pl.when(j >= 3)
            def _():
                for c in range(NSLOT):
                    @pl.when(lax.rem(j, NSLOT) == c)
                    def _(c=c):
                        xout_copy(c, 0).wait()
            for c in range(NSLOT):
                @pl.when(lax.rem(cblk, NSLOT) == c)
                def _(c=c):
                    x_rdma(c).wait_recv()
                    xout_copy(c, cblk).start()

        @pl.when((j >= 1) & (j <= NBH))
        def _():
            b = j - 1
            for c in range(NSLOT):
                @pl.when(lax.rem(b, NSLOT) == c)
                def _(c=c):
                    @pl.when(b >= NSLOT)
                    def _():
                        x_rdma(c).wait_send()
                        outd_copy(c, 0).wait()
                    y_rdma(c).wait_recv()
                    for aslot in range(2):
                        @pl.when(lax.rem(b, 2) == aslot)
                        def _(aslot=aslot):
                            sum_ref[c] = acc_ref[aslot] + yrecv_ref[c]
                    outd_copy(c, b).start()
                    x_rdma(c).start()

        @pl.when(j == NBH + 1)
        def _():
            for c in range(NSLOT):
                y_rdma(c).wait_send()
                x_rdma(c).wait_send()
                outd_copy(c, 0).wait()
            xout_copy(lax.rem(NBH - 1, NSLOT), 0).wait()

    return pl.pallas_call(
        body,
        grid=(NBH + 2,),
        in_specs=[
            pl.BlockSpec(memory_space=pl.ANY),
            pl.BlockSpec(memory_space=pl.ANY),
        ],
        out_specs=pl.BlockSpec(memory_space=pl.ANY),
        out_shape=jax.ShapeDtypeStruct((1, S_HALF, n), jnp.float32),
        scratch_shapes=[
            pltpu.VMEM((S, k), jnp.float32),
            pltpu.VMEM((2, S_HALF, BN), jnp.float32),
            pltpu.VMEM((2, k, BN), jnp.float32),
            pltpu.VMEM((NSLOT, S_HALF, BN), jnp.float32),
            pltpu.VMEM((NSLOT, S_HALF, BN), jnp.float32),
            pltpu.VMEM((NSLOT, S_HALF, BN), jnp.float32),
            pltpu.VMEM((NSLOT, S_HALF, BN), jnp.float32),
            pltpu.SemaphoreType.DMA((32,)),
            pltpu.SemaphoreType.DMA((NSLOT,)),
            pltpu.SemaphoreType.DMA((NSLOT,)),
            pltpu.SemaphoreType.DMA((NSLOT,)),
            pltpu.SemaphoreType.DMA((NSLOT,)),
            pltpu.SemaphoreType.DMA((2,)),
            pltpu.SemaphoreType.DMA((NSLOT,)),
            pltpu.SemaphoreType.DMA((NSLOT,)),
        ],
        compiler_params=pltpu.CompilerParams(
            collective_id=0,
            dimension_semantics=("arbitrary",),
            vmem_limit_bytes=64 * 1024 * 1024,
        ),
    )(O, Wo)


# device time: 262174 ns/iter; 1.6928x vs baseline; 1.0005x over previous
import jax
import jax.numpy as jnp
from jax import lax
from jax.experimental import pallas as pl
from jax.experimental.pallas import tpu as pltpu

N = 8192
NBH = 16
BN = (N // 2) // NBH
S = 2048
S_HALF = S // 2
NSLOT = 4


def kernel(O, Wo):
    _, s, nh, d = O.shape
    k, n = Wo.shape
    assert (s, nh * d, n) == (S, k, N)

    def body(o_ref, wo_ref, out_ref, o2_ref, acc_ref, wbuf_ref, ysend_ref,
             yrecv_ref, sum_ref, xrecv_ref, odma_sems, ysend_sems,
             yrecv_sems, xsend_sems, xrecv_sems, wdma_sems, outd_sems,
             xout_sems):
        j = pl.program_id(0)
        my_x = lax.axis_index("x")
        my_y = lax.axis_index("y")
        ypeer = (my_x, 1 - my_y)
        xpeer = (1 - my_x, my_y)
        my_start = my_y * S_HALF
        other_start = (1 - my_y) * S_HALF
        my_col0 = my_x * (N // 2)
        other_col0 = (1 - my_x) * (N // 2)

        def y_rdma(c):
            return pltpu.make_async_remote_copy(
                src_ref=ysend_ref.at[c], dst_ref=yrecv_ref.at[c],
                send_sem=ysend_sems.at[c], recv_sem=yrecv_sems.at[c],
                device_id=ypeer, device_id_type=pl.DeviceIdType.MESH)

        def x_rdma(c):
            return pltpu.make_async_remote_copy(
                src_ref=sum_ref.at[c], dst_ref=xrecv_ref.at[c],
                send_sem=xsend_sems.at[c], recv_sem=xrecv_sems.at[c],
                device_id=xpeer, device_id_type=pl.DeviceIdType.MESH)

        def w_copy(slot, blk):
            return pltpu.make_async_copy(
                wo_ref.at[:, pl.ds(my_col0 + blk * BN, BN)],
                wbuf_ref.at[slot], wdma_sems.at[slot])

        def outd_copy(c, b):
            return pltpu.make_async_copy(
                sum_ref.at[c],
                out_ref.at[0, :, pl.ds(my_col0 + b * BN, BN)],
                outd_sems.at[c])

        def xout_copy(c, b):
            return pltpu.make_async_copy(
                xrecv_ref.at[c],
                out_ref.at[0, :, pl.ds(other_col0 + b * BN, BN)],
                xout_sems.at[c])

        def o_copy(hh):
            return pltpu.make_async_copy(
                o_ref.at[0, :, hh, :],
                o2_ref.at[:, pl.ds(hh * d, d)],
                odma_sems.at[hh])

        @pl.when(j == 0)
        def _():
            for hh in range(nh):
                o_copy(hh).start()
            barrier = pltpu.get_barrier_semaphore()
            for nbr in (ypeer, xpeer):
                pl.semaphore_signal(barrier, inc=1, device_id=nbr,
                                    device_id_type=pl.DeviceIdType.MESH)
            pl.semaphore_wait(barrier, 2)
            w_copy(0, 0).start()
            for hh in range(nh):
                o_copy(hh).wait()

        @pl.when(j < NBH)
        def _():
            for wslot in range(2):
                @pl.when(lax.rem(j, 2) == wslot)
                def _(wslot=wslot):
                    w_copy(wslot, j).wait()
                    @pl.when(j + 1 < NBH)
                    def _():
                        w_copy(1 - wslot, j + 1).start()
                    for c in range(NSLOT):
                        @pl.when(lax.rem(j, NSLOT) == c)
                        def _(c=c):
                            @pl.when(j >= NSLOT)
                            def _():
                                y_rdma(c).wait_send()
                            ysend_ref[c] = jnp.dot(
                                o2_ref[pl.ds(other_start, S_HALF), :],
                                wbuf_ref[wslot],
                                preferred_element_type=jnp.float32)
                            y_rdma(c).start()
                    acc_ref[lax.rem(j, NSLOT)] = jnp.dot(
                        o2_ref[pl.ds(my_start, S_HALF), :], wbuf_ref[wslot],
                        preferred_element_type=jnp.float32)

        @pl.when(j >= 3)
        def _():
            cblk = j - 3
            @pl.when(j >= 4)
            def _():
                for c in range(NSLOT):
                    @pl.when(lax.rem(j, NSLOT) == c)
                    def _(c=c):
                        xout_copy(c, 0).wait()
            for c in range(NSLOT):
                @pl.when(lax.rem(cblk, NSLOT) == c)
                def _(c=c):
                    x_rdma(c).wait_recv()
                    xout_copy(c, cblk).start()

        @pl.when((j >= 2) & (j <= NBH + 1))
        def _():
            b = j - 2
            for c in range(NSLOT):
                @pl.when(lax.rem(b, NSLOT) == c)
                def _(c=c):
                    @pl.when(b >= NSLOT)
                    def _():
                        x_rdma(c).wait_send()
                        outd_copy(c, 0).wait()
                    y_rdma(c).wait_recv()
                    sum_ref[c] = acc_ref[lax.rem(b, NSLOT)] + yrecv_ref[c]
                    outd_copy(c, b).start()
                    x_rdma(c).start()

        @pl.when(j == NBH + 2)
        def _():
            for c in range(NSLOT):
                y_rdma(c).wait_send()
                x_rdma(c).wait_send()
                outd_copy(c, 0).wait()
            xout_copy((NBH - 1) % NSLOT, 0).wait()

    return pl.pallas_call(
        body,
        grid=(NBH + 3,),
        in_specs=[
            pl.BlockSpec(memory_space=pl.ANY),
            pl.BlockSpec(memory_space=pl.ANY),
        ],
        out_specs=pl.BlockSpec(memory_space=pl.ANY),
        out_shape=jax.ShapeDtypeStruct((1, S_HALF, n), jnp.float32),
        scratch_shapes=[
            pltpu.VMEM((S, k), jnp.float32),
            pltpu.VMEM((NSLOT, S_HALF, BN), jnp.float32),
            pltpu.VMEM((2, k, BN), jnp.float32),
            pltpu.VMEM((NSLOT, S_HALF, BN), jnp.float32),
            pltpu.VMEM((NSLOT, S_HALF, BN), jnp.float32),
            pltpu.VMEM((NSLOT, S_HALF, BN), jnp.float32),
            pltpu.VMEM((NSLOT, S_HALF, BN), jnp.float32),
            pltpu.SemaphoreType.DMA((32,)),
            pltpu.SemaphoreType.DMA((NSLOT,)),
            pltpu.SemaphoreType.DMA((NSLOT,)),
            pltpu.SemaphoreType.DMA((NSLOT,)),
            pltpu.SemaphoreType.DMA((NSLOT,)),
            pltpu.SemaphoreType.DMA((2,)),
            pltpu.SemaphoreType.DMA((NSLOT,)),
            pltpu.SemaphoreType.DMA((NSLOT,)),
        ],
        compiler_params=pltpu.CompilerParams(
            collective_id=0,
            dimension_semantics=("arbitrary",),
            vmem_limit_bytes=64 * 1024 * 1024,
        ),
    )(O, Wo)
